# baseline (speedup 1.0000x reference)
import functools

import jax
import jax.numpy as jnp
from jax import lax
from jax.experimental import pallas as pl
from jax.experimental.pallas import tpu as pltpu
from jax.experimental.pallas import tpu_sc as plsc

_N_GRAPHS = 16


def _mm_key_body(p_ref, x_ref, b_ref, o_ref):
    xp = lax.dot_general(
        p_ref[...], x_ref[...], (((0,), (1,)), ((), ())),
        preferred_element_type=jnp.float32,
    )
    u = lax.bitcast_convert_type(xp, jnp.uint32)
    neg = (u >> jnp.uint32(31)) == jnp.uint32(1)
    mono = jnp.where(neg, ~u, u ^ jnp.uint32(0x80000000))
    bat = b_ref[...].astype(jnp.uint32)
    key_u = (bat << jnp.uint32(28)) | (mono >> jnp.uint32(4))
    o_ref[...] = lax.bitcast_convert_type(key_u ^ jnp.uint32(0x80000000),
                                          jnp.int32)


def _make_keys(x, batch, projections, row_blk=4096, col_blk=128):
    n, c = x.shape
    pr = projections.shape[1]
    grid = (pr // col_blk, n // row_blk)
    return pl.pallas_call(
        _mm_key_body,
        grid=grid,
        in_specs=[
            pl.BlockSpec((c, col_blk), lambda j, i: (0, j)),
            pl.BlockSpec((row_blk, c), lambda j, i: (i, 0)),
            pl.BlockSpec((1, row_blk), lambda j, i: (0, i)),
        ],
        out_specs=pl.BlockSpec((col_blk, row_blk), lambda j, i: (j, i)),
        out_shape=jax.ShapeDtypeStruct((pr, n), jnp.int32),
    )(projections, x, batch.reshape(1, n))


def _cex(a, down, up, lobit, dirbit):
    partner = jnp.where(lobit == 0, down, up)
    take_min = (lobit ^ dirbit) == 0
    return jnp.where(take_min, jnp.minimum(a, partner),
                     jnp.maximum(a, partner))


def _sort_body(n_exp, in_ref, o_ref):
    o_ref[...] = in_ref[...]
    cb, rows, _ = o_ref.shape
    ig = (lax.broadcasted_iota(jnp.int32, (cb, rows, 128), 1) * 128
          + lax.broadcasted_iota(jnp.int32, (cb, rows, 128), 2))

    def level(k_exp, _):
        @pl.when(k_exp >= 8)
        def _():
            dirbit = (ig >> k_exp) & 1

            def sub_stage(t, a):
                j_exp = k_exp - 1 - t
                j = jnp.int32(1) << (j_exp - 7)
                down = pltpu.roll(a, rows - j, 1)
                up = pltpu.roll(a, j, 1)
                return _cex(a, down, up, (ig >> j_exp) & 1, dirbit)

            o_ref[...] = lax.fori_loop(jnp.int32(0), k_exp - 7, sub_stage,
                                       o_ref[...])

        cr = min(8, rows)

        def chunk_body(c, _):
            ch = o_ref[:, pl.ds(c * cr, cr), :]
            ig_ch = ((lax.broadcasted_iota(jnp.int32, ch.shape, 1) + c * cr)
                     * 128
                     + lax.broadcasted_iota(jnp.int32, ch.shape, 2))
            dirbit = (ig_ch >> k_exp) & 1

            def lane_stage(t, ch):
                j_exp = jnp.minimum(k_exp - 1, 6) - t
                j = jnp.int32(1) << j_exp
                down = pltpu.roll(ch, 128 - j, 2)
                up = pltpu.roll(ch, j, 2)
                return _cex(ch, down, up, (ig_ch >> j_exp) & 1, dirbit)

            ch = lax.fori_loop(jnp.int32(0), jnp.minimum(k_exp, 7),
                               lane_stage, ch)
            o_ref[:, pl.ds(c * cr, cr), :] = ch
            return 0

        lax.fori_loop(jnp.int32(0), jnp.int32(rows // cr), chunk_body, 0)
        return 0

    lax.fori_loop(jnp.int32(1), jnp.int32(n_exp + 1), level, 0)


def _sort_keys(keys_t, col_blk=8):
    pr, n = keys_t.shape
    n_exp = n.bit_length() - 1
    assert (1 << n_exp) == n and n % 128 == 0
    rows = n // 128
    k3 = keys_t.reshape(pr, rows, 128)
    out = pl.pallas_call(
        functools.partial(_sort_body, n_exp),
        grid=(pr // col_blk,),
        in_specs=[pl.BlockSpec((col_blk, rows, 128), lambda j: (j, 0, 0))],
        out_specs=pl.BlockSpec((col_blk, rows, 128), lambda j: (j, 0, 0)),
        out_shape=jax.ShapeDtypeStruct((pr, rows, 128), jnp.int32),
    )(k3)
    return out.reshape(pr, n)


def _sc_gather(flat_table, flat_idx):
    info = plsc.get_sparse_core_info()
    nw = info.num_cores * info.num_subcores
    b = flat_idx.shape[0]
    b_per_w = b // nw
    mesh = plsc.VectorSubcoreMesh(core_axis_name="c", subcore_axis_name="s")

    @functools.partial(
        pl.kernel,
        mesh=mesh,
        out_type=jax.ShapeDtypeStruct((b,), jnp.int32),
        scratch_types=[
            pltpu.VMEM((b_per_w,), jnp.int32),
            pltpu.VMEM((b_per_w,), jnp.int32),
            pltpu.SemaphoreType.DMA,
        ],
    )
    def k(table_hbm, idx_hbm, out_hbm, idx_v, vals_v, sem):
        wid = lax.axis_index("s") * info.num_cores + lax.axis_index("c")
        base = wid * b_per_w
        pltpu.sync_copy(idx_hbm.at[pl.ds(base, b_per_w)], idx_v)
        pltpu.async_copy(table_hbm.at[idx_v], vals_v, sem).wait()
        pltpu.sync_copy(vals_v, out_hbm.at[pl.ds(base, b_per_w)])

    return k(flat_table, flat_idx)


def _decode(keys_i32):
    ku = lax.bitcast_convert_type(keys_i32, jnp.uint32) ^ jnp.uint32(0x80000000)
    u = (ku & jnp.uint32(0x0FFFFFFF)) << jnp.uint32(4)
    nonneg = u >= jnp.uint32(0x80000000)
    bits = jnp.where(nonneg, u ^ jnp.uint32(0x80000000), ~u)
    return lax.bitcast_convert_type(bits, jnp.float32)


def kernel(x, batch, projections, cum_weights):
    n, _ = x.shape
    pr = projections.shape[1]
    q = cum_weights.shape[0]
    g = _N_GRAPHS

    keys_t = _make_keys(x, batch, projections)
    sorted_t = _sort_keys(keys_t)

    counts = jnp.bincount(batch, length=g)
    starts = jnp.cumsum(counts) - counts
    scaled = cum_weights[None, :] * (counts[:, None] - 1).astype(cum_weights.dtype)
    qidx = jnp.floor(scaled).astype(jnp.int32)
    gather_idx = starts[:, None].astype(jnp.int32) + qidx
    tok = jnp.clip(gather_idx.reshape(-1), 0, n - 1)

    flat_idx = (jnp.arange(pr, dtype=jnp.int32)[None, :] * n
                + tok[:, None]).reshape(-1)
    rows = _sc_gather(sorted_t.reshape(-1), flat_idx)
    vals = _decode(rows)
    out = vals.reshape(g, q * pr)
    return out / float((q * pr) ** 0.5)


# speedup vs baseline: 2.3539x; 2.3539x over previous
import functools

import jax
import jax.numpy as jnp
from jax import lax
from jax.experimental import pallas as pl
from jax.experimental.pallas import tpu as pltpu
from jax.experimental.pallas import tpu_sc as plsc

_N_GRAPHS = 16


def _mm_key_body(p_ref, x_ref, b_ref, o_ref):
    xp = lax.dot_general(
        p_ref[...], x_ref[...], (((0,), (1,)), ((), ())),
        preferred_element_type=jnp.float32,
    )
    u = lax.bitcast_convert_type(xp, jnp.uint32)
    neg = (u >> jnp.uint32(31)) == jnp.uint32(1)
    mono = jnp.where(neg, ~u, u ^ jnp.uint32(0x80000000))
    bat = b_ref[...].astype(jnp.uint32)
    key_u = (bat << jnp.uint32(28)) | (mono >> jnp.uint32(4))
    o_ref[...] = lax.bitcast_convert_type(key_u ^ jnp.uint32(0x80000000),
                                          jnp.int32)


def _make_keys(x, batch, projections, row_blk=4096, col_blk=128):
    n, c = x.shape
    pr = projections.shape[1]
    grid = (pr // col_blk, n // row_blk)
    return pl.pallas_call(
        _mm_key_body,
        grid=grid,
        in_specs=[
            pl.BlockSpec((c, col_blk), lambda j, i: (0, j)),
            pl.BlockSpec((row_blk, c), lambda j, i: (i, 0)),
            pl.BlockSpec((1, row_blk), lambda j, i: (0, i)),
        ],
        out_specs=pl.BlockSpec((col_blk, row_blk), lambda j, i: (j, i)),
        out_shape=jax.ShapeDtypeStruct((pr, n), jnp.int32),
    )(projections, x, batch.reshape(1, n))


def _cex(a, down, up, lobit, dirbit):
    partner = jnp.where(lobit == 0, down, up)
    take_min = (lobit ^ dirbit) == 0
    return jnp.where(take_min, jnp.minimum(a, partner),
                     jnp.maximum(a, partner))


def _sort_body(n_exp, in_ref, o_ref):
    a = in_ref[...]
    rows = a.shape[1]
    ig = (lax.broadcasted_iota(jnp.int32, a.shape, 1) * 128
          + lax.broadcasted_iota(jnp.int32, a.shape, 2))

    def level(k_exp, a):
        dirbit = (ig >> k_exp) & 1

        def sub_stage(t, a):
            j_exp = k_exp - 1 - t
            j = jnp.int32(1) << (j_exp - 7)
            down = pltpu.roll(a, rows - j, 1)
            up = pltpu.roll(a, j, 1)
            return _cex(a, down, up, (ig >> j_exp) & 1, dirbit)

        def lane_stage(t, a):
            j_exp = jnp.minimum(k_exp - 1, 6) - t
            j = jnp.int32(1) << j_exp
            down = pltpu.roll(a, 128 - j, 2)
            up = pltpu.roll(a, j, 2)
            return _cex(a, down, up, (ig >> j_exp) & 1, dirbit)

        a = lax.fori_loop(jnp.int32(0), jnp.maximum(k_exp - 7, 0),
                          sub_stage, a)
        a = lax.fori_loop(jnp.int32(0), jnp.minimum(k_exp, 7),
                          lane_stage, a)
        return a

    o_ref[...] = lax.fori_loop(jnp.int32(1), jnp.int32(n_exp + 1), level, a)


def _sort_keys(keys_t, col_blk=8):
    pr, n = keys_t.shape
    n_exp = n.bit_length() - 1
    assert (1 << n_exp) == n and n % 128 == 0
    rows = n // 128
    k3 = keys_t.reshape(pr, rows, 128)
    out = pl.pallas_call(
        functools.partial(_sort_body, n_exp),
        grid=(pr // col_blk,),
        in_specs=[pl.BlockSpec((col_blk, rows, 128), lambda j: (j, 0, 0))],
        out_specs=pl.BlockSpec((col_blk, rows, 128), lambda j: (j, 0, 0)),
        out_shape=jax.ShapeDtypeStruct((pr, rows, 128), jnp.int32),
    )(k3)
    return out.reshape(pr, n)


def _sc_gather(flat_table, flat_idx):
    info = plsc.get_sparse_core_info()
    nw = info.num_cores * info.num_subcores
    b = flat_idx.shape[0]
    b_per_w = b // nw
    mesh = plsc.VectorSubcoreMesh(core_axis_name="c", subcore_axis_name="s")

    @functools.partial(
        pl.kernel,
        mesh=mesh,
        out_type=jax.ShapeDtypeStruct((b,), jnp.int32),
        scratch_types=[
            pltpu.VMEM((b_per_w,), jnp.int32),
            pltpu.VMEM((b_per_w,), jnp.int32),
            pltpu.SemaphoreType.DMA,
        ],
    )
    def k(table_hbm, idx_hbm, out_hbm, idx_v, vals_v, sem):
        wid = lax.axis_index("s") * info.num_cores + lax.axis_index("c")
        base = wid * b_per_w
        pltpu.sync_copy(idx_hbm.at[pl.ds(base, b_per_w)], idx_v)
        pltpu.async_copy(table_hbm.at[idx_v], vals_v, sem).wait()
        pltpu.sync_copy(vals_v, out_hbm.at[pl.ds(base, b_per_w)])

    return k(flat_table, flat_idx)


def _decode(keys_i32):
    ku = lax.bitcast_convert_type(keys_i32, jnp.uint32) ^ jnp.uint32(0x80000000)
    u = (ku & jnp.uint32(0x0FFFFFFF)) << jnp.uint32(4)
    nonneg = u >= jnp.uint32(0x80000000)
    bits = jnp.where(nonneg, u ^ jnp.uint32(0x80000000), ~u)
    return lax.bitcast_convert_type(bits, jnp.float32)


def kernel(x, batch, projections, cum_weights):
    n, _ = x.shape
    pr = projections.shape[1]
    q = cum_weights.shape[0]
    g = _N_GRAPHS

    keys_t = _make_keys(x, batch, projections)
    sorted_t = _sort_keys(keys_t)

    counts = jnp.bincount(batch, length=g)
    starts = jnp.cumsum(counts) - counts
    scaled = cum_weights[None, :] * (counts[:, None] - 1).astype(cum_weights.dtype)
    qidx = jnp.floor(scaled).astype(jnp.int32)
    gather_idx = starts[:, None].astype(jnp.int32) + qidx
    tok = jnp.clip(gather_idx.reshape(-1), 0, n - 1)

    flat_idx = (jnp.arange(pr, dtype=jnp.int32)[None, :] * n
                + tok[:, None]).reshape(-1)
    rows = _sc_gather(sorted_t.reshape(-1), flat_idx)
    vals = _decode(rows)
    out = vals.reshape(g, q * pr)
    return out / float((q * pr) ** 0.5)
